# Initial kernel scaffold; baseline (speedup 1.0000x reference)
import jax
import jax.numpy as jnp
from jax.experimental import pallas as pl


def kernel(boxes_logits, cls_logits, priors):
    raise NotImplementedError("write your pallas kernel here")


# speedup vs baseline: 243.9579x; 243.9579x over previous
import functools
import jax
import jax.numpy as jnp
from jax import lax
from jax.experimental import pallas as pl
from jax.experimental.pallas import tpu as pltpu

B = 8
N = 20000
TOP_K = 750
IOU_THRESH = 0.01
CONF_THRESH = 0.3
V0, V1 = 0.1, 0.2

ROWS = 157
NPAD = ROWS * 128
OUTL = 768
NEG_INF = float("-inf")


def _nms_body(l0, l1, lx, ly, lw, lh, px, py, pw, ph,
              ox1, oy1, ox2, oy2, osc, ocnt,
              sw_s, x1o_s, y1o_s, x2o_s, y2o_s, area_s):
    a0 = l0[0]
    a1 = l1[0]
    m01 = jnp.maximum(a0, a1)
    e0 = jnp.exp(a0 - m01)
    e1 = jnp.exp(a1 - m01)
    sc = e1 / (e0 + e1)

    cx = px[...] + lx[0] * V0 * pw[...]
    cy = py[...] + ly[0] * V0 * ph[...]
    w = pw[...] * jnp.exp(lw[0] * V1)
    h = ph[...] * jnp.exp(lh[0] * V1)
    x1 = (cx - w / 2.0) * 1024.0
    y1 = (cy - h / 2.0) * 1024.0
    x2 = ((cx - w / 2.0) + w) * 1024.0
    y2 = ((cy - h / 2.0) + h) * 1024.0

    row_i = lax.broadcasted_iota(jnp.int32, (ROWS, 128), 0)
    col_i = lax.broadcasted_iota(jnp.int32, (ROWS, 128), 1)
    fiota = row_i * 128 + col_i
    valid = fiota < N
    mask = (sc > CONF_THRESH) & valid

    cmax = jnp.maximum(jnp.maximum(x1, y1), jnp.maximum(x2, y2))
    mc = jnp.max(jnp.where(mask, cmax, NEG_INF))
    finite = (mc == mc) & (jnp.abs(mc) != jnp.inf)
    off = jnp.where(finite, mc, 0.0) + 1.0

    x1o = x1 + off
    y1o = y1 + off
    x2o = x2 + off
    y2o = y2 + off
    area = (x2o - x1o) * (y2o - y1o)

    sw_s[...] = jnp.where(mask, sc, NEG_INF)
    x1o_s[...] = x1o
    y1o_s[...] = y1o
    x2o_s[...] = x2o
    y2o_s[...] = y2o
    area_s[...] = area

    zer = jnp.zeros((1, 1, OUTL), jnp.float32)
    ox1[...] = zer
    oy1[...] = zer
    ox2[...] = zer
    oy2[...] = zer
    osc[...] = zer

    lane_out = lax.broadcasted_iota(jnp.int32, (1, 1, OUTL), 2)
    lane128 = lax.broadcasted_iota(jnp.int32, (1, 128), 1)

    def cond(state):
        cnt, active = state
        return active

    def body(state):
        cnt, active = state
        sw = sw_s[...]
        m = jnp.max(sw)
        keep = m > NEG_INF
        idx = jnp.min(jnp.where(sw == m, fiota, jnp.int32(2**31 - 1)))
        q = idx // 128
        r = idx - q * 128

        ohr = lane128 == r

        def pick(ref):
            row = ref[pl.ds(q, 1), :]
            return jnp.sum(jnp.where(ohr, row, 0.0))

        kx1 = pick(x1o_s)
        ky1 = pick(y1o_s)
        kx2 = pick(x2o_s)
        ky2 = pick(y2o_s)
        kar = pick(area_s)

        xx1 = jnp.maximum(kx1, x1o_s[...])
        yy1 = jnp.maximum(ky1, y1o_s[...])
        xx2 = jnp.minimum(kx2, x2o_s[...])
        yy2 = jnp.minimum(ky2, y2o_s[...])
        iw = jnp.maximum(xx2 - xx1, 0.0)
        ih = jnp.maximum(yy2 - yy1, 0.0)
        inter = iw * ih
        iou = inter / (kar + area_s[...] - inter)
        supp = (iou > IOU_THRESH) | (fiota == idx)
        sw_s[...] = jnp.where(supp & keep, NEG_INF, sw)

        oh = (lane_out == cnt) & keep
        ox1[...] = jnp.where(oh, kx1 - off, ox1[...])
        oy1[...] = jnp.where(oh, ky1 - off, oy1[...])
        ox2[...] = jnp.where(oh, kx2 - off, ox2[...])
        oy2[...] = jnp.where(oh, ky2 - off, oy2[...])
        osc[...] = jnp.where(oh, m, osc[...])

        cnt2 = cnt + keep.astype(jnp.int32)
        active2 = keep & (cnt2 < TOP_K)
        return cnt2, active2

    cnt, _ = lax.while_loop(cond, body, (jnp.int32(0), jnp.bool_(True)))
    ocnt[...] = jnp.broadcast_to(cnt, (1, 1, 128)).astype(jnp.int32)


@jax.jit
def kernel(boxes_logits, cls_logits, priors):
    pad = NPAD - N

    def prep(x):
        return jnp.pad(x, ((0, 0), (0, pad))).reshape(B, ROWS, 128)

    def prep1(x):
        return jnp.pad(x, ((0, pad),)).reshape(ROWS, 128)

    l0 = prep(cls_logits[:, :, 0])
    l1 = prep(cls_logits[:, :, 1])
    lx = prep(boxes_logits[:, :, 0])
    ly = prep(boxes_logits[:, :, 1])
    lw = prep(boxes_logits[:, :, 2])
    lh = prep(boxes_logits[:, :, 3])
    px = prep1(priors[:, 0])
    py = prep1(priors[:, 1])
    pw = prep1(priors[:, 2])
    ph = prep1(priors[:, 3])

    bspec = pl.BlockSpec((1, ROWS, 128), lambda b: (b, 0, 0))
    pspec = pl.BlockSpec((ROWS, 128), lambda b: (0, 0))
    ospec = pl.BlockSpec((1, 1, OUTL), lambda b: (b, 0, 0))
    cspec = pl.BlockSpec((1, 1, 128), lambda b: (b, 0, 0))

    out = pl.pallas_call(
        _nms_body,
        grid=(B,),
        in_specs=[bspec] * 6 + [pspec] * 4,
        out_specs=[ospec] * 5 + [cspec],
        out_shape=[jax.ShapeDtypeStruct((B, 1, OUTL), jnp.float32)] * 5
        + [jax.ShapeDtypeStruct((B, 1, 128), jnp.int32)],
        scratch_shapes=[pltpu.VMEM((ROWS, 128), jnp.float32)] * 6,
    )(l0, l1, lx, ly, lw, lh, px, py, pw, ph)

    ox1, oy1, ox2, oy2, osc, ocnt = out
    pred_boxes = jnp.stack(
        [ox1[:, 0, :TOP_K], oy1[:, 0, :TOP_K], ox2[:, 0, :TOP_K], oy2[:, 0, :TOP_K]],
        axis=-1,
    )
    pred_scores = osc[:, 0, :TOP_K]
    counts = ocnt[:, 0, 0]
    slot = jnp.arange(TOP_K, dtype=jnp.int32)[None, :]
    pred_labels = jnp.where(slot < counts[:, None], 1, 0).astype(jnp.int64)
    return pred_boxes, pred_scores, pred_labels, counts


# speedup vs baseline: 740.9854x; 3.0373x over previous
import functools
import jax
import jax.numpy as jnp
from jax import lax
from jax.experimental import pallas as pl
from jax.experimental.pallas import tpu as pltpu
from jax.experimental.pallas import tpu_sc as plsc

B = 8
N = 20000
TOP_K = 750
IOU_THRESH = 0.01
CONF_THRESH = 0.3
V0, V1 = 0.1, 0.2

CH = 2000
NCHUNKS = N // CH
VPC = CH // 16
LCAP = N + 16
OUTL = 768
NEG_INF = float("-inf")


def _sc_body(l0_h, l1_h, lx_h, ly_h, lw_h, lh_h, px_h, py_h, pw_h, ph_h,
             ox1_h, oy1_h, ox2_h, oy2_h, osc_h, ocnt_h,
             s0, s1, sx, sy, sw_, sh_, spx, spy, spw, sph,
             L1, L2, L3, L4, L5,
             O1, O2, O3, O4, O5, OC, sem):
    wid = lax.axis_index("s") * 2 + lax.axis_index("c")

    @pl.when(wid < B)
    def _work():
        b = wid
        base = b * N
        lane = lax.broadcasted_iota(jnp.int32, (16,), 0)
        zero16 = jnp.zeros((16,), jnp.float32)

        def zbody(i, _):
            O1[pl.ds(i * 16, 16)] = zero16
            O2[pl.ds(i * 16, 16)] = zero16
            O3[pl.ds(i * 16, 16)] = zero16
            O4[pl.ds(i * 16, 16)] = zero16
            O5[pl.ds(i * 16, 16)] = zero16
            return 0
        lax.fori_loop(0, OUTL // 16, zbody, 0)

        def chunk(c, carry):
            w, mcv = carry
            o = base + c * CH
            po = c * CH
            cps = [
                pltpu.async_copy(l0_h.at[pl.ds(o, CH)], s0, sem),
                pltpu.async_copy(l1_h.at[pl.ds(o, CH)], s1, sem),
                pltpu.async_copy(lx_h.at[pl.ds(o, CH)], sx, sem),
                pltpu.async_copy(ly_h.at[pl.ds(o, CH)], sy, sem),
                pltpu.async_copy(lw_h.at[pl.ds(o, CH)], sw_, sem),
                pltpu.async_copy(lh_h.at[pl.ds(o, CH)], sh_, sem),
                pltpu.async_copy(px_h.at[pl.ds(po, CH)], spx, sem),
                pltpu.async_copy(py_h.at[pl.ds(po, CH)], spy, sem),
                pltpu.async_copy(pw_h.at[pl.ds(po, CH)], spw, sem),
                pltpu.async_copy(ph_h.at[pl.ds(po, CH)], sph, sem),
            ]
            for cp in cps:
                cp.wait()

            def vec(i, carry2):
                w2, mcv2 = carry2
                sl = pl.ds(i * 16, 16)
                a0 = s0[sl]
                a1 = s1[sl]
                m01 = jnp.maximum(a0, a1)
                e0 = jnp.exp(a0 - m01)
                e1 = jnp.exp(a1 - m01)
                sc = e1 / (e0 + e1)

                pxv = spx[sl]
                pyv = spy[sl]
                pwv = spw[sl]
                phv = sph[sl]
                cx = pxv + sx[sl] * V0 * pwv
                cy = pyv + sy[sl] * V0 * phv
                bw = pwv * jnp.exp(sw_[sl] * V1)
                bh = phv * jnp.exp(sh_[sl] * V1)
                tx = cx - bw / 2.0
                ty = cy - bh / 2.0
                x1 = tx * 1024.0
                y1 = ty * 1024.0
                x2 = (tx + bw) * 1024.0
                y2 = (ty + bh) * 1024.0

                m = sc > CONF_THRESH
                cmax = jnp.maximum(jnp.maximum(x1, y1), jnp.maximum(x2, y2))
                mcv3 = jnp.maximum(mcv2, jnp.where(m, cmax, NEG_INF))

                dst = pl.ds(w2, 16)
                plsc.store_compressed(L1.at[dst], x1, mask=m)
                plsc.store_compressed(L2.at[dst], y1, mask=m)
                plsc.store_compressed(L3.at[dst], x2, mask=m)
                plsc.store_compressed(L4.at[dst], y2, mask=m)
                plsc.store_compressed(L5.at[dst], sc, mask=m)
                w3 = w2 + jnp.sum(m.astype(jnp.int32))
                return w3, mcv3

            return lax.fori_loop(0, VPC, vec, (w, mcv))

        nlive, mcv = lax.fori_loop(
            0, NCHUNKS, chunk,
            (jnp.int32(0), jnp.full((16,), NEG_INF, jnp.float32)))

        mc = jnp.max(mcv)
        finite = (mc == mc) & (jnp.abs(mc) != jnp.inf)
        off = jnp.where(finite, mc, 0.0) + 1.0

        nch0 = (nlive + 15) // 16

        def offb(i, _):
            sl = pl.ds(i * 16, 16)
            L1[sl] = L1[sl] + off
            L2[sl] = L2[sl] + off
            L3[sl] = L3[sl] + off
            L4[sl] = L4[sl] + off
            return 0
        lax.fori_loop(0, nch0, offb, 0)

        def scanb(i, carry):
            bms, b1, b2, b3, b4, bpos = carry
            p = i * 16
            sl = pl.ds(p, 16)
            scv = L5[sl]
            vmask = (p + lane) < nlive
            smk = jnp.where(vmask, scv, NEG_INF)
            chm = jnp.max(smk)
            is_new = chm > bms
            ln = jnp.min(jnp.where(smk == chm, lane, 16))
            selm = lane == ln
            ex = lambda v: jnp.max(jnp.where(selm, v, NEG_INF))
            n1 = jnp.where(is_new, ex(L1[sl]), b1)
            n2 = jnp.where(is_new, ex(L2[sl]), b2)
            n3 = jnp.where(is_new, ex(L3[sl]), b3)
            n4 = jnp.where(is_new, ex(L4[sl]), b4)
            nms = jnp.where(is_new, chm, bms)
            npos = jnp.where(is_new, p + ln, bpos)
            return nms, n1, n2, n3, n4, npos

        best0 = lax.fori_loop(
            0, nch0, scanb,
            (jnp.float32(NEG_INF), jnp.float32(0), jnp.float32(0),
             jnp.float32(0), jnp.float32(0), jnp.int32(0)))

        def cond(st):
            cnt, nlive_, bms, b1, b2, b3, b4, bpos = st
            return (bms > NEG_INF) & (cnt < TOP_K)

        def body(st):
            cnt, nlive_, bms, b1, b2, b3, b4, bpos = st
            cm = lane == 0
            iv = jnp.full((16,), cnt, jnp.int32)
            plsc.store_scatter(O1, [iv], jnp.full((16,), b1 - off), mask=cm)
            plsc.store_scatter(O2, [iv], jnp.full((16,), b2 - off), mask=cm)
            plsc.store_scatter(O3, [iv], jnp.full((16,), b3 - off), mask=cm)
            plsc.store_scatter(O4, [iv], jnp.full((16,), b4 - off), mask=cm)
            plsc.store_scatter(O5, [iv], jnp.full((16,), bms), mask=cm)
            cnt2 = cnt + 1

            karea = (b3 - b1) * (b4 - b2)
            nch = (nlive_ + 15) // 16

            def passb(i, carry):
                w, nbms, n1, n2, n3, n4, npos = carry
                p = i * 16
                sl = pl.ds(p, 16)
                x1v = L1[sl]
                y1v = L2[sl]
                x2v = L3[sl]
                y2v = L4[sl]
                scv = L5[sl]
                vmask = (p + lane) < nlive_
                areav = (x2v - x1v) * (y2v - y1v)
                xx1 = jnp.maximum(b1, x1v)
                yy1 = jnp.maximum(b2, y1v)
                xx2 = jnp.minimum(b3, x2v)
                yy2 = jnp.minimum(b4, y2v)
                iw = jnp.maximum(xx2 - xx1, 0.0)
                ih = jnp.maximum(yy2 - yy1, 0.0)
                inter = iw * ih
                iou = inter / (karea + areav - inter)
                keepm = (~(iou > IOU_THRESH)) & vmask & ((p + lane) != bpos)

                dst = pl.ds(w, 16)
                plsc.store_compressed(L1.at[dst], x1v, mask=keepm)
                plsc.store_compressed(L2.at[dst], y1v, mask=keepm)
                plsc.store_compressed(L3.at[dst], x2v, mask=keepm)
                plsc.store_compressed(L4.at[dst], y2v, mask=keepm)
                plsc.store_compressed(L5.at[dst], scv, mask=keepm)
                nk = jnp.sum(keepm.astype(jnp.int32))

                smk = jnp.where(keepm, scv, NEG_INF)
                chm = jnp.max(smk)
                is_new = chm > nbms
                ln = jnp.min(jnp.where(smk == chm, lane, 16))
                selm = lane == ln
                ex = lambda v: jnp.max(jnp.where(selm, v, NEG_INF))
                rank = jnp.sum((keepm & (lane < ln)).astype(jnp.int32))
                m1 = jnp.where(is_new, ex(x1v), n1)
                m2 = jnp.where(is_new, ex(y1v), n2)
                m3 = jnp.where(is_new, ex(x2v), n3)
                m4 = jnp.where(is_new, ex(y2v), n4)
                mms = jnp.where(is_new, chm, nbms)
                mpos = jnp.where(is_new, w + rank, npos)
                return w + nk, mms, m1, m2, m3, m4, mpos

            res = lax.fori_loop(
                0, nch, passb,
                (jnp.int32(0), jnp.float32(NEG_INF), jnp.float32(0),
                 jnp.float32(0), jnp.float32(0), jnp.float32(0),
                 jnp.int32(0)))
            w, nbms, n1, n2, n3, n4, npos = res
            return cnt2, w, nbms, n1, n2, n3, n4, npos

        st = lax.while_loop(
            cond, body,
            (jnp.int32(0), nlive, best0[0], best0[1], best0[2], best0[3],
             best0[4], best0[5]))
        cnt_final = st[0]

        OC[...] = jnp.full((16,), cnt_final, jnp.int32)
        pltpu.sync_copy(O1, ox1_h.at[pl.ds(b * OUTL, OUTL)])
        pltpu.sync_copy(O2, oy1_h.at[pl.ds(b * OUTL, OUTL)])
        pltpu.sync_copy(O3, ox2_h.at[pl.ds(b * OUTL, OUTL)])
        pltpu.sync_copy(O4, oy2_h.at[pl.ds(b * OUTL, OUTL)])
        pltpu.sync_copy(O5, osc_h.at[pl.ds(b * OUTL, OUTL)])
        pltpu.sync_copy(OC, ocnt_h.at[pl.ds(b * 16, 16)])


_mesh = plsc.VectorSubcoreMesh(
    core_axis_name="c", subcore_axis_name="s", num_cores=2, num_subcores=16)

_sc_call = functools.partial(
    pl.kernel,
    out_type=[jax.ShapeDtypeStruct((B * OUTL,), jnp.float32)] * 5
    + [jax.ShapeDtypeStruct((B * 16,), jnp.int32)],
    mesh=_mesh,
    scratch_types=[pltpu.VMEM((CH,), jnp.float32)] * 10
    + [pltpu.VMEM((LCAP,), jnp.float32)] * 5
    + [pltpu.VMEM((OUTL,), jnp.float32)] * 5
    + [pltpu.VMEM((16,), jnp.int32), pltpu.SemaphoreType.DMA],
    compiler_params=pltpu.CompilerParams(needs_layout_passes=False),
)(_sc_body)


@jax.jit
def kernel(boxes_logits, cls_logits, priors):
    l0 = cls_logits[:, :, 0].reshape(-1)
    l1 = cls_logits[:, :, 1].reshape(-1)
    lx = boxes_logits[:, :, 0].reshape(-1)
    ly = boxes_logits[:, :, 1].reshape(-1)
    lw = boxes_logits[:, :, 2].reshape(-1)
    lh = boxes_logits[:, :, 3].reshape(-1)
    px = priors[:, 0]
    py = priors[:, 1]
    pw = priors[:, 2]
    ph = priors[:, 3]

    ox1, oy1, ox2, oy2, osc, ocnt = _sc_call(
        l0, l1, lx, ly, lw, lh, px, py, pw, ph)

    ox1 = ox1.reshape(B, OUTL)
    oy1 = oy1.reshape(B, OUTL)
    ox2 = ox2.reshape(B, OUTL)
    oy2 = oy2.reshape(B, OUTL)
    osc = osc.reshape(B, OUTL)
    pred_boxes = jnp.stack(
        [ox1[:, :TOP_K], oy1[:, :TOP_K], ox2[:, :TOP_K], oy2[:, :TOP_K]],
        axis=-1,
    )
    pred_scores = osc[:, :TOP_K]
    counts = ocnt.reshape(B, 16)[:, 0]
    slot = jnp.arange(TOP_K, dtype=jnp.int32)[None, :]
    pred_labels = jnp.where(slot < counts[:, None], 1, 0).astype(jnp.int64)
    return pred_boxes, pred_scores, pred_labels, counts


# speedup vs baseline: 742.7801x; 1.0024x over previous
import functools
import jax
import jax.numpy as jnp
from jax import lax
from jax.experimental import pallas as pl
from jax.experimental.pallas import tpu as pltpu
from jax.experimental.pallas import tpu_sc as plsc

B = 8
N = 20000
TOP_K = 750
IOU_THRESH = 0.01
CONF_THRESH = 0.3
V0, V1 = 0.1, 0.2

CH = 2000
NCHUNKS = N // CH
VPC = CH // 16
LCAP = N + 16
OUTL = 768
NEG_INF = float("-inf")


def _sc_body(l0_h, l1_h, lx_h, ly_h, lw_h, lh_h, px_h, py_h, pw_h, ph_h,
             ox1_h, oy1_h, ox2_h, oy2_h, osc_h, ocnt_h,
             s0, s1, sx, sy, sw_, sh_, spx, spy, spw, sph,
             L1, L2, L3, L4, L5,
             O1, O2, O3, O4, O5, OC, sem):
    wid = lax.axis_index("s") * 2 + lax.axis_index("c")

    @pl.when(wid < B)
    def _work():
        b = wid
        base = b * N
        lane = lax.broadcasted_iota(jnp.int32, (16,), 0)
        zero16 = jnp.zeros((16,), jnp.float32)

        def zbody(i, _):
            O1[pl.ds(i * 16, 16)] = zero16
            O2[pl.ds(i * 16, 16)] = zero16
            O3[pl.ds(i * 16, 16)] = zero16
            O4[pl.ds(i * 16, 16)] = zero16
            O5[pl.ds(i * 16, 16)] = zero16
            return 0
        lax.fori_loop(0, OUTL // 16, zbody, 0)

        def chunk(c, carry):
            w, mcv = carry
            o = base + c * CH
            po = c * CH
            cps = [
                pltpu.async_copy(l0_h.at[pl.ds(o, CH)], s0, sem),
                pltpu.async_copy(l1_h.at[pl.ds(o, CH)], s1, sem),
                pltpu.async_copy(lx_h.at[pl.ds(o, CH)], sx, sem),
                pltpu.async_copy(ly_h.at[pl.ds(o, CH)], sy, sem),
                pltpu.async_copy(lw_h.at[pl.ds(o, CH)], sw_, sem),
                pltpu.async_copy(lh_h.at[pl.ds(o, CH)], sh_, sem),
                pltpu.async_copy(px_h.at[pl.ds(po, CH)], spx, sem),
                pltpu.async_copy(py_h.at[pl.ds(po, CH)], spy, sem),
                pltpu.async_copy(pw_h.at[pl.ds(po, CH)], spw, sem),
                pltpu.async_copy(ph_h.at[pl.ds(po, CH)], sph, sem),
            ]
            for cp in cps:
                cp.wait()

            def vec(i, carry2):
                w2, mcv2 = carry2
                sl = pl.ds(i * 16, 16)
                a0 = s0[sl]
                a1 = s1[sl]
                m01 = jnp.maximum(a0, a1)
                e0 = jnp.exp(a0 - m01)
                e1 = jnp.exp(a1 - m01)
                sc = e1 / (e0 + e1)

                pxv = spx[sl]
                pyv = spy[sl]
                pwv = spw[sl]
                phv = sph[sl]
                cx = pxv + sx[sl] * V0 * pwv
                cy = pyv + sy[sl] * V0 * phv
                bw = pwv * jnp.exp(sw_[sl] * V1)
                bh = phv * jnp.exp(sh_[sl] * V1)
                tx = cx - bw / 2.0
                ty = cy - bh / 2.0
                x1 = tx * 1024.0
                y1 = ty * 1024.0
                x2 = (tx + bw) * 1024.0
                y2 = (ty + bh) * 1024.0

                m = sc > CONF_THRESH
                cmax = jnp.maximum(jnp.maximum(x1, y1), jnp.maximum(x2, y2))
                mcv3 = jnp.maximum(mcv2, jnp.where(m, cmax, NEG_INF))

                dst = pl.ds(w2, 16)
                plsc.store_compressed(L1.at[dst], x1, mask=m)
                plsc.store_compressed(L2.at[dst], y1, mask=m)
                plsc.store_compressed(L3.at[dst], x2, mask=m)
                plsc.store_compressed(L4.at[dst], y2, mask=m)
                plsc.store_compressed(L5.at[dst], sc, mask=m)
                w3 = w2 + jnp.sum(m.astype(jnp.int32))
                return w3, mcv3

            return lax.fori_loop(0, VPC, vec, (w, mcv))

        nlive, mcv = lax.fori_loop(
            0, NCHUNKS, chunk,
            (jnp.int32(0), jnp.full((16,), NEG_INF, jnp.float32)))

        mc = jnp.max(mcv)
        finite = (mc == mc) & (jnp.abs(mc) != jnp.inf)
        off = jnp.where(finite, mc, 0.0) + 1.0

        nch0 = (nlive + 15) // 16

        def scanb(i, carry):
            bms, b1, b2, b3, b4, bpos = carry
            p = i * 16
            sl = pl.ds(p, 16)
            x1v = L1[sl] + off
            y1v = L2[sl] + off
            x2v = L3[sl] + off
            y2v = L4[sl] + off
            L1[sl] = x1v
            L2[sl] = y1v
            L3[sl] = x2v
            L4[sl] = y2v
            scv = L5[sl]
            vmask = (p + lane) < nlive
            smk = jnp.where(vmask, scv, NEG_INF)
            chm = jnp.max(smk)
            is_new = chm > bms

            def newf(args):
                x1v, y1v, x2v, y2v = args
                ln = jnp.min(jnp.where(smk == chm, lane, 16))
                selm = lane == ln
                ex = lambda v: jnp.max(jnp.where(selm, v, NEG_INF))
                return (chm, ex(x1v), ex(y1v), ex(x2v), ex(y2v), p + ln)

            return lax.cond(
                is_new, newf,
                lambda args: (bms, b1, b2, b3, b4, bpos),
                (x1v, y1v, x2v, y2v))

        best0 = lax.fori_loop(
            0, nch0, scanb,
            (jnp.float32(NEG_INF), jnp.float32(0), jnp.float32(0),
             jnp.float32(0), jnp.float32(0), jnp.int32(0)))

        def cond(st):
            cnt, nlive_, bms, b1, b2, b3, b4, bpos = st
            return (bms > NEG_INF) & (cnt < TOP_K)

        def body(st):
            cnt, nlive_, bms, b1, b2, b3, b4, bpos = st
            cm = lane == 0
            iv = jnp.full((16,), cnt, jnp.int32)
            plsc.store_scatter(O1, [iv], jnp.full((16,), b1 - off), mask=cm)
            plsc.store_scatter(O2, [iv], jnp.full((16,), b2 - off), mask=cm)
            plsc.store_scatter(O3, [iv], jnp.full((16,), b3 - off), mask=cm)
            plsc.store_scatter(O4, [iv], jnp.full((16,), b4 - off), mask=cm)
            plsc.store_scatter(O5, [iv], jnp.full((16,), bms), mask=cm)
            cnt2 = cnt + 1

            karea = (b3 - b1) * (b4 - b2)
            nch = (nlive_ + 31) // 32

            def sub(p, carry):
                w, nbms, n1, n2, n3, n4, npos = carry
                sl = pl.ds(p, 16)
                x1v = L1[sl]
                y1v = L2[sl]
                x2v = L3[sl]
                y2v = L4[sl]
                scv = L5[sl]
                vmask = (p + lane) < nlive_
                areav = (x2v - x1v) * (y2v - y1v)
                xx1 = jnp.maximum(b1, x1v)
                yy1 = jnp.maximum(b2, y1v)
                xx2 = jnp.minimum(b3, x2v)
                yy2 = jnp.minimum(b4, y2v)
                iw = jnp.maximum(xx2 - xx1, 0.0)
                ih = jnp.maximum(yy2 - yy1, 0.0)
                inter = iw * ih
                iou = inter / (karea + areav - inter)
                keepm = (~(iou > IOU_THRESH)) & vmask & ((p + lane) != bpos)

                dst = pl.ds(w, 16)
                plsc.store_compressed(L1.at[dst], x1v, mask=keepm)
                plsc.store_compressed(L2.at[dst], y1v, mask=keepm)
                plsc.store_compressed(L3.at[dst], x2v, mask=keepm)
                plsc.store_compressed(L4.at[dst], y2v, mask=keepm)
                plsc.store_compressed(L5.at[dst], scv, mask=keepm)
                nk = jnp.sum(keepm.astype(jnp.int32))

                smk = jnp.where(keepm, scv, NEG_INF)
                chm = jnp.max(smk)
                is_new = chm > nbms

                def newf(args):
                    x1v, y1v, x2v, y2v, keepm = args
                    ln = jnp.min(jnp.where(smk == chm, lane, 16))
                    selm = lane == ln
                    ex = lambda v: jnp.max(jnp.where(selm, v, NEG_INF))
                    rank = jnp.sum((keepm & (lane < ln)).astype(jnp.int32))
                    return (chm, ex(x1v), ex(y1v), ex(x2v), ex(y2v), w + rank)

                nbms, n1, n2, n3, n4, npos = lax.cond(
                    is_new, newf,
                    lambda args: (nbms, n1, n2, n3, n4, npos),
                    (x1v, y1v, x2v, y2v, keepm))
                return w + nk, nbms, n1, n2, n3, n4, npos

            def passb(i, carry):
                carry = sub(i * 32, carry)
                carry = sub(i * 32 + 16, carry)
                return carry

            res = lax.fori_loop(
                0, nch, passb,
                (jnp.int32(0), jnp.float32(NEG_INF), jnp.float32(0),
                 jnp.float32(0), jnp.float32(0), jnp.float32(0),
                 jnp.int32(0)))
            w, nbms, n1, n2, n3, n4, npos = res
            return cnt2, w, nbms, n1, n2, n3, n4, npos

        st = lax.while_loop(
            cond, body,
            (jnp.int32(0), nlive, best0[0], best0[1], best0[2], best0[3],
             best0[4], best0[5]))
        cnt_final = st[0]

        OC[...] = jnp.full((16,), cnt_final, jnp.int32)
        pltpu.sync_copy(O1, ox1_h.at[pl.ds(b * OUTL, OUTL)])
        pltpu.sync_copy(O2, oy1_h.at[pl.ds(b * OUTL, OUTL)])
        pltpu.sync_copy(O3, ox2_h.at[pl.ds(b * OUTL, OUTL)])
        pltpu.sync_copy(O4, oy2_h.at[pl.ds(b * OUTL, OUTL)])
        pltpu.sync_copy(O5, osc_h.at[pl.ds(b * OUTL, OUTL)])
        pltpu.sync_copy(OC, ocnt_h.at[pl.ds(b * 16, 16)])


_mesh = plsc.VectorSubcoreMesh(
    core_axis_name="c", subcore_axis_name="s", num_cores=2, num_subcores=16)

_sc_call = functools.partial(
    pl.kernel,
    out_type=[jax.ShapeDtypeStruct((B * OUTL,), jnp.float32)] * 5
    + [jax.ShapeDtypeStruct((B * 16,), jnp.int32)],
    mesh=_mesh,
    scratch_types=[pltpu.VMEM((CH,), jnp.float32)] * 10
    + [pltpu.VMEM((LCAP,), jnp.float32)] * 5
    + [pltpu.VMEM((OUTL,), jnp.float32)] * 5
    + [pltpu.VMEM((16,), jnp.int32), pltpu.SemaphoreType.DMA],
    compiler_params=pltpu.CompilerParams(needs_layout_passes=False),
)(_sc_body)


@jax.jit
def kernel(boxes_logits, cls_logits, priors):
    l0 = cls_logits[:, :, 0].reshape(-1)
    l1 = cls_logits[:, :, 1].reshape(-1)
    lx = boxes_logits[:, :, 0].reshape(-1)
    ly = boxes_logits[:, :, 1].reshape(-1)
    lw = boxes_logits[:, :, 2].reshape(-1)
    lh = boxes_logits[:, :, 3].reshape(-1)
    px = priors[:, 0]
    py = priors[:, 1]
    pw = priors[:, 2]
    ph = priors[:, 3]

    ox1, oy1, ox2, oy2, osc, ocnt = _sc_call(
        l0, l1, lx, ly, lw, lh, px, py, pw, ph)

    ox1 = ox1.reshape(B, OUTL)
    oy1 = oy1.reshape(B, OUTL)
    ox2 = ox2.reshape(B, OUTL)
    oy2 = oy2.reshape(B, OUTL)
    osc = osc.reshape(B, OUTL)
    pred_boxes = jnp.stack(
        [ox1[:, :TOP_K], oy1[:, :TOP_K], ox2[:, :TOP_K], oy2[:, :TOP_K]],
        axis=-1,
    )
    pred_scores = osc[:, :TOP_K]
    counts = ocnt.reshape(B, 16)[:, 0]
    slot = jnp.arange(TOP_K, dtype=jnp.int32)[None, :]
    pred_labels = jnp.where(slot < counts[:, None], 1, 0).astype(jnp.int64)
    return pred_boxes, pred_scores, pred_labels, counts
